# Initial kernel scaffold; baseline (speedup 1.0000x reference)
import jax
import jax.numpy as jnp
from jax.experimental import pallas as pl


def kernel(x, edge_index, enc_W, enc_b, enc_gamma, enc_beta, conv_W, conv_b, bn_gamma, bn_beta, dec_W1, dec_b1, dec_W2, dec_b2, dec_W3, dec_b3):
    raise NotImplementedError("write your pallas kernel here")


# speedup vs baseline: 2.1445x; 2.1445x over previous
import functools

import jax
import jax.numpy as jnp
from jax.experimental import pallas as pl
from jax.experimental.pallas import tpu as pltpu

N = 10000
N_PAD = 10240
E = 320000
H = 128
L = 6
EPS = 1e-5
INV_N = 1.0 / N


def _bn_relu(h, gamma, beta):
    mu = jnp.sum(h, axis=0, keepdims=True) * INV_N
    d = h - mu
    var = jnp.sum(d * d, axis=0, keepdims=True) * INV_N
    return jax.nn.relu(d * jax.lax.rsqrt(var + EPS) * gamma + beta)


def _encoder_body(x_ref, w_ref, b_ref, g_ref, be_ref, deg_ref, w0_ref,
                  h_ref, dinv_ref, msc_ref):
    x = x_ref[...]
    h = jax.nn.relu(
        jnp.dot(x, w_ref[...], preferred_element_type=jnp.float32)
        + b_ref[...])
    mu = jnp.sum(h, axis=0, keepdims=True) * INV_N
    d = h - mu
    var = jnp.sum(d * d, axis=0, keepdims=True) * INV_N
    h = d * jax.lax.rsqrt(var + EPS) * g_ref[...] + be_ref[...]
    h_ref[...] = h
    deg = deg_ref[...]
    dinv = jnp.where(deg > 0, jax.lax.rsqrt(jnp.maximum(deg, 1.0)), 0.0)
    dinv_ref[...] = dinv
    m = jnp.dot(h, w0_ref[...], preferred_element_type=jnp.float32)
    msc_ref[0:N, :] = m * dinv[0:N]
    msc_ref[N:N_PAD, :] = jnp.zeros((N_PAD - N, H), jnp.float32)


def _layer_body(h_ref, p0_ref, p1_ref, dinv_ref, b_ref, g_ref, be_ref,
                wn_ref, ho_ref, msc_ref):
    dinv = dinv_ref[...]
    agg = (p0_ref[0:N, :] + p1_ref[0:N, :]) * dinv[0:N]
    hn = _bn_relu(agg + b_ref[...], g_ref[...], be_ref[...])
    h = h_ref[...] + hn
    ho_ref[...] = h
    m = jnp.dot(h, wn_ref[...], preferred_element_type=jnp.float32)
    msc_ref[0:N, :] = m * dinv[0:N]
    msc_ref[N:N_PAD, :] = jnp.zeros((N_PAD - N, H), jnp.float32)


def _final_body(h_ref, p0_ref, p1_ref, dinv_ref, b_ref, g_ref, be_ref,
                w1_ref, b1_ref, w2_ref, b2_ref, w3_ref, b3_ref, o_ref):
    dinv = dinv_ref[...]
    agg = (p0_ref[0:N, :] + p1_ref[0:N, :]) * dinv[0:N]
    hn = _bn_relu(agg + b_ref[...], g_ref[...], be_ref[...])
    h = h_ref[...] + hn
    o = jax.nn.relu(
        jnp.dot(h, w1_ref[...], preferred_element_type=jnp.float32)
        + b1_ref[...])
    o = jax.nn.relu(
        jnp.dot(o, w2_ref[...], preferred_element_type=jnp.float32)
        + b2_ref[...])
    o_ref[...] = (
        jnp.dot(o, w3_ref[...], preferred_element_type=jnp.float32)
        + b3_ref[...])


def kernel(x, edge_index, enc_W, enc_b, enc_gamma, enc_beta, conv_W, conv_b,
           bn_gamma, bn_beta, dec_W1, dec_b1, dec_W2, dec_b2, dec_W3, dec_b3):
    src = edge_index[0]
    dst = edge_index[1]

    deg = jnp.zeros((N,), jnp.float32).at[dst].add(1.0) + 1.0
    deg_col = jnp.zeros((N_PAD, 1), jnp.float32).at[0:N, 0].set(deg)

    x8 = jnp.zeros((N, 8), jnp.float32).at[:, 0:4].set(x)
    w8 = jnp.zeros((8, H), jnp.float32).at[0:4, :].set(enc_W)

    h, dinv, msc = pl.pallas_call(
        _encoder_body,
        out_shape=[
            jax.ShapeDtypeStruct((N, H), jnp.float32),
            jax.ShapeDtypeStruct((N_PAD, 1), jnp.float32),
            jax.ShapeDtypeStruct((N_PAD, H), jnp.float32),
        ],
    )(x8, w8, enc_b, enc_gamma, enc_beta, deg_col, conv_W[0])

    zeros_part = jnp.zeros((N_PAD, H), jnp.float32)

    for i in range(L):
        part0 = msc.at[dst].add(msc[src])
        if i < L - 1:
            h, msc = pl.pallas_call(
                _layer_body,
                out_shape=[
                    jax.ShapeDtypeStruct((N, H), jnp.float32),
                    jax.ShapeDtypeStruct((N_PAD, H), jnp.float32),
                ],
            )(h, part0, zeros_part, dinv, conv_b[i], bn_gamma[i], bn_beta[i],
              conv_W[i + 1])
        else:
            o = pl.pallas_call(
                _final_body,
                out_shape=jax.ShapeDtypeStruct((N, 1), jnp.float32),
            )(h, part0, zeros_part, dinv, conv_b[i], bn_gamma[i], bn_beta[i],
              dec_W1, dec_b1, dec_W2, dec_b2, dec_W3, dec_b3)
    return o[:, 0]
